# baseline (speedup 1.0000x reference)
import functools

import jax
import jax.numpy as jnp
from jax import lax
from jax.experimental import pallas as pl
from jax.experimental.pallas import tpu as pltpu
from jax.experimental.pallas import tpu_sc as plsc

N = 2048
SD = 64
HM = 128
N_STEPS = 10
CAP = 40960
NC = 2
NS = 16
NT = NC * NS
EPT = CAP // NT
CHUNK = 128
NCH = EPT // CHUNK
DEG = 64
NSLOT = N * DEG
SPT = NSLOT // NT
NCH2 = SPT // CHUNK

_f32 = jnp.float32


def _sc_gather_body(a_hbm, b_hbm, ridx_hbm, cidx_hbm, ar_out, bc_out,
                    ridx_v, cidx_v, bufr0, bufr1, bufc0, bufc1, sem0, sem1):
    wid = lax.axis_index("s") * NC + lax.axis_index("c")
    base = wid * EPT
    pltpu.sync_copy(ridx_hbm.at[wid], ridx_v)
    pltpu.sync_copy(cidx_hbm.at[wid], cidx_v)
    bufr = (bufr0, bufr1)
    bufc = (bufc0, bufc1)
    sem = (sem0, sem1)
    pend = [None, None]

    def drain(j):
        cr, cc = pend[j & 1]
        cr.wait()
        cc.wait()
        off = base + j * CHUNK
        pltpu.sync_copy(bufr[j & 1], ar_out.at[pl.ds(off, CHUNK)])
        pltpu.sync_copy(bufc[j & 1], bc_out.at[pl.ds(off, CHUNK)])

    for j in range(NCH):
        p = j & 1
        cr = pltpu.async_copy(a_hbm.at[ridx_v.at[j]], bufr[p], sem[p])
        cc = pltpu.async_copy(b_hbm.at[cidx_v.at[j]], bufc[p], sem[p])
        pend[p] = (cr, cc)
        if j > 0:
            drain(j - 1)
    drain(NCH - 1)


def _sc_pgather_body(msgs_hbm, ptr_hbm, out_hbm, ptr_v, buf0, buf1, sem0, sem1):
    wid = lax.axis_index("s") * NC + lax.axis_index("c")
    base = wid * SPT
    pltpu.sync_copy(ptr_hbm.at[wid], ptr_v)

    def pair(k, carry):
        j0 = 2 * k
        c0 = pltpu.async_copy(msgs_hbm.at[ptr_v.at[j0]], buf0, sem0)
        c1 = pltpu.async_copy(msgs_hbm.at[ptr_v.at[j0 + 1]], buf1, sem1)
        c0.wait()
        pltpu.sync_copy(buf0, out_hbm.at[pl.ds(base + j0 * CHUNK, CHUNK)])
        c1.wait()
        pltpu.sync_copy(buf1, out_hbm.at[pl.ds(base + (j0 + 1) * CHUNK, CHUNK)])
        return carry

    lax.fori_loop(0, NCH2 // 2, pair, 0)


def _make_sc_calls():
    mesh = plsc.VectorSubcoreMesh(core_axis_name="c", subcore_axis_name="s",
                                  num_cores=NC, num_subcores=NS)
    gather = pl.kernel(
        _sc_gather_body,
        out_type=(jax.ShapeDtypeStruct((CAP, HM), _f32),
                  jax.ShapeDtypeStruct((CAP, HM), _f32)),
        mesh=mesh,
        scratch_types=[
            pltpu.VMEM((NCH, CHUNK), jnp.int32),
            pltpu.VMEM((NCH, CHUNK), jnp.int32),
            pltpu.VMEM((CHUNK, HM), _f32),
            pltpu.VMEM((CHUNK, HM), _f32),
            pltpu.VMEM((CHUNK, HM), _f32),
            pltpu.VMEM((CHUNK, HM), _f32),
            pltpu.SemaphoreType.DMA,
            pltpu.SemaphoreType.DMA,
        ],
    )
    pgather = pl.kernel(
        _sc_pgather_body,
        out_type=jax.ShapeDtypeStruct((NSLOT, HM), _f32),
        mesh=mesh,
        scratch_types=[
            pltpu.VMEM((NCH2, CHUNK), jnp.int32),
            pltpu.VMEM((CHUNK, HM), _f32),
            pltpu.VMEM((CHUNK, HM), _f32),
            pltpu.SemaphoreType.DMA,
            pltpu.SemaphoreType.DMA,
        ],
    )
    return gather, pgather


EB = 2560


def _mlp_body(hr2, hc2, ef, w1, b1, w2, b2, w3, b3, out):
    lane = lax.broadcasted_iota(jnp.int32, (EB, HM), 1)
    xpre = jnp.where(lane < SD, hr2[...], hc2[...])
    xcat = jnp.concatenate([xpre, ef[...]], axis=1)
    x = jnp.maximum(jnp.dot(xcat, w1[...], preferred_element_type=_f32) + b1[...], 0.0)
    x = jnp.maximum(jnp.dot(x, w2[...], preferred_element_type=_f32) + b2[...], 0.0)
    out[...] = (jnp.dot(x, w3[...], preferred_element_type=_f32) + b3[...]) * ef[:, 4:5]


NB = 256


def _reduce_body(p, out):
    acc = p[0]
    for j in range(1, DEG):
        acc = acc + p[j]
    out[...] = acc


def _gru_body(nm, h2, wih, bih, whh, bhh, out):
    x = nm[:, :SD]
    hh = h2[:, :SD]
    gi = jnp.dot(x, wih[...], preferred_element_type=_f32) + bih[...]
    gh = jnp.dot(hh, whh[...], preferred_element_type=_f32) + bhh[...]
    r = jax.nn.sigmoid(gi[:, 0:SD] + gh[:, 0:SD])
    z = jax.nn.sigmoid(gi[:, SD:2 * SD] + gh[:, SD:2 * SD])
    n = jnp.tanh(gi[:, 2 * SD:] + r * gh[:, 2 * SD:])
    hnew = (1.0 - z) * n + z * hh
    out[...] = jnp.concatenate([hnew, hnew], axis=1)


def _readout_body(h, w1, b1, w2, b2, wd, bd, out):
    x = jnp.maximum(jnp.dot(h[...], w1[...], preferred_element_type=_f32) + b1[...], 0.0)
    x = jnp.maximum(jnp.dot(x, w2[...], preferred_element_type=_f32) + b2[...], 0.0)
    d = jnp.sum(x * wd[...], axis=1, keepdims=True) + bd[...]
    sgn = 1.0 - 2.0 * lax.broadcasted_iota(jnp.int32, (N, 2), 1).astype(_f32)
    out[...] = jax.nn.sigmoid(sgn * d)


def _make_tc_calls():
    full = pl.BlockSpec(index_map=lambda i: (0, 0))
    mlp = pl.pallas_call(
        _mlp_body,
        grid=(CAP // EB,),
        in_specs=[
            pl.BlockSpec((EB, HM), lambda i: (i, 0)),
            pl.BlockSpec((EB, HM), lambda i: (i, 0)),
            pl.BlockSpec((EB, 8), lambda i: (i, 0)),
            full, full, full, full, full, full,
        ],
        out_specs=pl.BlockSpec((EB, HM), lambda i: (i, 0)),
        out_shape=jax.ShapeDtypeStruct((CAP, HM), _f32),
    )
    reduce = pl.pallas_call(
        _reduce_body,
        grid=(N // NB,),
        in_specs=[pl.BlockSpec((DEG, NB, HM), lambda i: (0, i, 0))],
        out_specs=pl.BlockSpec((NB, HM), lambda i: (i, 0)),
        out_shape=jax.ShapeDtypeStruct((N, HM), _f32),
    )
    gru = pl.pallas_call(
        _gru_body,
        out_shape=jax.ShapeDtypeStruct((N, HM), _f32),
    )
    readout = pl.pallas_call(
        _readout_body,
        out_shape=jax.ShapeDtypeStruct((N, 2), _f32),
    )
    return mlp, reduce, gru, readout


def kernel(J, b, W_m1, b_m1, W_m2, b_m2, W_m3, b_m3, W_ih, b_ih, W_hh, b_hh,
           W_r1, b_r1, W_r2, b_r2, W_r3, b_r3):
    flat = J.reshape(-1)
    (eidx,) = jnp.nonzero(flat, size=CAP, fill_value=0)
    cnt = jnp.count_nonzero(flat)
    ar = jnp.arange(CAP)
    valid = ar < cnt
    row0 = (eidx // N).astype(jnp.int32)
    col0 = (eidx - row0 * N).astype(jnp.int32)
    key = jnp.where(valid, col0 * N + row0, N * N + ar)
    perm = jnp.argsort(key)
    row = row0[perm]
    col = col0[perm]
    ei = eidx[perm]
    vf = valid.astype(_f32)[:, None]
    ef = jnp.stack([b[row], b[col], flat[ei], J[col, row],
                    valid.astype(_f32),
                    jnp.zeros(CAP, _f32), jnp.zeros(CAP, _f32),
                    jnp.zeros(CAP, _f32)], axis=-1) * vf
    spread = (ar % 128).astype(jnp.int32)
    row_g = jnp.where(valid, row, spread * 16)
    col_g = jnp.where(valid, col, spread * 16)
    ridx3 = row_g.reshape(NT, NCH, CHUNK)
    cidx3 = col_g.reshape(NT, NCH, CHUNK)

    segcnt = jnp.bincount(jnp.where(valid, col, N), length=N + 1)[:N]
    offs = jnp.concatenate([jnp.zeros((1,), segcnt.dtype), jnp.cumsum(segcnt)[:-1]])
    rank = ar - offs[col]
    slotpos = jnp.where(valid & (rank < DEG), rank * N + col, NSLOT)
    n_inval = jnp.maximum(CAP - cnt, 1)
    pad_ptr = (cnt + (jnp.arange(NSLOT) % n_inval)).astype(jnp.int32)
    ptr_flat = pad_ptr.at[slotpos].set(ar.astype(jnp.int32), mode="drop")
    ptr3 = ptr_flat.reshape(NT, NCH2, CHUNK)

    w1 = jnp.pad(W_m1.T, ((0, 4), (0, 0)))
    b1 = b_m1.reshape(1, HM)
    w2 = W_m2.T
    b2 = b_m2.reshape(1, HM)
    w3 = jnp.pad(W_m3.T, ((0, 0), (0, HM - SD)))
    b3 = jnp.pad(b_m3.reshape(1, SD), ((0, 0), (0, HM - SD)))
    wih = W_ih.T
    bih = b_ih.reshape(1, -1)
    whh = W_hh.T
    bhh = b_hh.reshape(1, -1)
    wr1 = W_r1.T
    br1 = b_r1.reshape(1, -1)
    wr2 = W_r2.T
    br2 = b_r2.reshape(1, -1)
    wd = (W_r3[0] - W_r3[1]).reshape(1, -1)
    bd = (b_r3[0] - b_r3[1]).reshape(1, 1)

    sc_gather, sc_pgather = _make_sc_calls()
    mlp, reduce, gru, readout = _make_tc_calls()

    def step(h2, _):
        hr2, hc2 = sc_gather(h2, h2, ridx3, cidx3)
        msgs = mlp(hr2, hc2, ef, w1, b1, w2, b2, w3, b3)
        slots = sc_pgather(msgs, ptr3)
        nm = reduce(slots.reshape(DEG, N, HM))
        h2 = gru(nm, h2, wih, bih, whh, bhh)
        return h2, None

    h2 = jnp.zeros((N, HM), _f32)
    h2, _ = lax.scan(step, h2, None, length=N_STEPS)
    return readout(h2[:, :SD], wr1, br1, wr2, br2, wd, bd)


# speedup vs baseline: 1.0077x; 1.0077x over previous
import functools

import jax
import jax.numpy as jnp
from jax import lax
from jax.experimental import pallas as pl
from jax.experimental.pallas import tpu as pltpu
from jax.experimental.pallas import tpu_sc as plsc

N = 2048
SD = 64
HM = 128
N_STEPS = 10
CAP = 40960
NC = 2
NS = 16
NT = NC * NS
EPT = CAP // NT
CHUNK = 128
NCH = EPT // CHUNK
DEG = 64
NSLOT = N * DEG
SPT = NSLOT // NT
NCH2 = SPT // CHUNK

_f32 = jnp.float32


def _sc_gather_body(a_hbm, b_hbm, ridx_hbm, cidx_hbm, ar_out, bc_out,
                    ridx_v, cidx_v, bufr0, bufr1, bufc0, bufc1, sem0, sem1):
    wid = lax.axis_index("s") * NC + lax.axis_index("c")
    base = wid * EPT
    pltpu.sync_copy(ridx_hbm.at[wid], ridx_v)
    pltpu.sync_copy(cidx_hbm.at[wid], cidx_v)
    bufr = (bufr0, bufr1)
    bufc = (bufc0, bufc1)
    sem = (sem0, sem1)
    pend = [None, None]

    def drain(j):
        cr, cc = pend[j & 1]
        cr.wait()
        cc.wait()
        off = base + j * CHUNK
        pltpu.sync_copy(bufr[j & 1], ar_out.at[pl.ds(off, CHUNK)])
        pltpu.sync_copy(bufc[j & 1], bc_out.at[pl.ds(off, CHUNK)])

    for j in range(NCH):
        p = j & 1
        cr = pltpu.async_copy(a_hbm.at[ridx_v.at[j]], bufr[p], sem[p])
        cc = pltpu.async_copy(b_hbm.at[cidx_v.at[j]], bufc[p], sem[p])
        pend[p] = (cr, cc)
        if j > 0:
            drain(j - 1)
    drain(NCH - 1)


def _sc_pgather_body(msgs_hbm, ptr_hbm, out_hbm, ptr_v,
                     buf0, buf1, buf2, buf3, sem0, sem1, sem2, sem3):
    wid = lax.axis_index("s") * NC + lax.axis_index("c")
    base = wid * SPT
    pltpu.sync_copy(ptr_hbm.at[wid], ptr_v)
    bufs = (buf0, buf1, buf2, buf3)
    sems = (sem0, sem1, sem2, sem3)

    def group(k, carry):
        j0 = 4 * k
        pend = [pltpu.async_copy(msgs_hbm.at[ptr_v.at[j0 + i]], bufs[i], sems[i])
                for i in range(4)]
        for i in range(4):
            pend[i].wait()
            pltpu.sync_copy(bufs[i], out_hbm.at[pl.ds(base + (j0 + i) * CHUNK, CHUNK)])
        return carry

    lax.fori_loop(0, NCH2 // 4, group, 0)


def _make_sc_calls():
    mesh = plsc.VectorSubcoreMesh(core_axis_name="c", subcore_axis_name="s",
                                  num_cores=NC, num_subcores=NS)
    gather = pl.kernel(
        _sc_gather_body,
        out_type=(jax.ShapeDtypeStruct((CAP, HM), _f32),
                  jax.ShapeDtypeStruct((CAP, HM), _f32)),
        mesh=mesh,
        scratch_types=[
            pltpu.VMEM((NCH, CHUNK), jnp.int32),
            pltpu.VMEM((NCH, CHUNK), jnp.int32),
            pltpu.VMEM((CHUNK, HM), _f32),
            pltpu.VMEM((CHUNK, HM), _f32),
            pltpu.VMEM((CHUNK, HM), _f32),
            pltpu.VMEM((CHUNK, HM), _f32),
            pltpu.SemaphoreType.DMA,
            pltpu.SemaphoreType.DMA,
        ],
    )
    pgather = pl.kernel(
        _sc_pgather_body,
        out_type=jax.ShapeDtypeStruct((NSLOT, HM), _f32),
        mesh=mesh,
        scratch_types=[
            pltpu.VMEM((NCH2, CHUNK), jnp.int32),
            pltpu.VMEM((CHUNK, HM), _f32),
            pltpu.VMEM((CHUNK, HM), _f32),
            pltpu.VMEM((CHUNK, HM), _f32),
            pltpu.VMEM((CHUNK, HM), _f32),
            pltpu.SemaphoreType.DMA,
            pltpu.SemaphoreType.DMA,
            pltpu.SemaphoreType.DMA,
            pltpu.SemaphoreType.DMA,
        ],
    )
    return gather, pgather


EB = 2560


def _mlp_body(hr2, hc2, ef, w1, b1, w2, b2, w3, b3, out):
    lane = lax.broadcasted_iota(jnp.int32, (EB, HM), 1)
    xpre = jnp.where(lane < SD, hr2[...], hc2[...])
    xcat = jnp.concatenate([xpre, ef[...]], axis=1)
    x = jnp.maximum(jnp.dot(xcat, w1[...], preferred_element_type=_f32) + b1[...], 0.0)
    x = jnp.maximum(jnp.dot(x, w2[...], preferred_element_type=_f32) + b2[...], 0.0)
    out[...] = (jnp.dot(x, w3[...], preferred_element_type=_f32) + b3[...]) * ef[:, 4:5]


NB = 256


def _reduce_body(p, out):
    acc = p[0]
    for j in range(1, DEG):
        acc = acc + p[j]
    out[...] = acc


def _gru_body(nm, h2, wih, bih, whh, bhh, out):
    x = nm[:, :SD]
    hh = h2[:, :SD]
    gi = jnp.dot(x, wih[...], preferred_element_type=_f32) + bih[...]
    gh = jnp.dot(hh, whh[...], preferred_element_type=_f32) + bhh[...]
    r = jax.nn.sigmoid(gi[:, 0:SD] + gh[:, 0:SD])
    z = jax.nn.sigmoid(gi[:, SD:2 * SD] + gh[:, SD:2 * SD])
    n = jnp.tanh(gi[:, 2 * SD:] + r * gh[:, 2 * SD:])
    hnew = (1.0 - z) * n + z * hh
    out[...] = jnp.concatenate([hnew, hnew], axis=1)


def _readout_body(h, w1, b1, w2, b2, wd, bd, out):
    x = jnp.maximum(jnp.dot(h[...], w1[...], preferred_element_type=_f32) + b1[...], 0.0)
    x = jnp.maximum(jnp.dot(x, w2[...], preferred_element_type=_f32) + b2[...], 0.0)
    d = jnp.sum(x * wd[...], axis=1, keepdims=True) + bd[...]
    sgn = 1.0 - 2.0 * lax.broadcasted_iota(jnp.int32, (N, 2), 1).astype(_f32)
    out[...] = jax.nn.sigmoid(sgn * d)


def _make_tc_calls():
    full = pl.BlockSpec(index_map=lambda i: (0, 0))
    mlp = pl.pallas_call(
        _mlp_body,
        grid=(CAP // EB,),
        in_specs=[
            pl.BlockSpec((EB, HM), lambda i: (i, 0)),
            pl.BlockSpec((EB, HM), lambda i: (i, 0)),
            pl.BlockSpec((EB, 8), lambda i: (i, 0)),
            full, full, full, full, full, full,
        ],
        out_specs=pl.BlockSpec((EB, HM), lambda i: (i, 0)),
        out_shape=jax.ShapeDtypeStruct((CAP, HM), _f32),
    )
    reduce = pl.pallas_call(
        _reduce_body,
        grid=(N // NB,),
        in_specs=[pl.BlockSpec((DEG, NB, HM), lambda i: (0, i, 0))],
        out_specs=pl.BlockSpec((NB, HM), lambda i: (i, 0)),
        out_shape=jax.ShapeDtypeStruct((N, HM), _f32),
    )
    gru = pl.pallas_call(
        _gru_body,
        out_shape=jax.ShapeDtypeStruct((N, HM), _f32),
    )
    readout = pl.pallas_call(
        _readout_body,
        out_shape=jax.ShapeDtypeStruct((N, 2), _f32),
    )
    return mlp, reduce, gru, readout


def kernel(J, b, W_m1, b_m1, W_m2, b_m2, W_m3, b_m3, W_ih, b_ih, W_hh, b_hh,
           W_r1, b_r1, W_r2, b_r2, W_r3, b_r3):
    flat = J.reshape(-1)
    (eidx,) = jnp.nonzero(flat, size=CAP, fill_value=0)
    cnt = jnp.count_nonzero(flat)
    ar = jnp.arange(CAP)
    valid = ar < cnt
    row0 = (eidx // N).astype(jnp.int32)
    col0 = (eidx - row0 * N).astype(jnp.int32)
    key = jnp.where(valid, col0 * N + row0, N * N + ar)
    perm = jnp.argsort(key)
    row = row0[perm]
    col = col0[perm]
    ei = eidx[perm]
    vf = valid.astype(_f32)[:, None]
    ef = jnp.stack([b[row], b[col], flat[ei], J[col, row],
                    valid.astype(_f32),
                    jnp.zeros(CAP, _f32), jnp.zeros(CAP, _f32),
                    jnp.zeros(CAP, _f32)], axis=-1) * vf
    spread = (ar % 128).astype(jnp.int32)
    row_g = jnp.where(valid, row, spread * 16)
    col_g = jnp.where(valid, col, spread * 16)
    ridx3 = row_g.reshape(NT, NCH, CHUNK)
    cidx3 = col_g.reshape(NT, NCH, CHUNK)

    segcnt = jnp.bincount(jnp.where(valid, col, N), length=N + 1)[:N]
    offs = jnp.concatenate([jnp.zeros((1,), segcnt.dtype), jnp.cumsum(segcnt)[:-1]])
    rank = ar - offs[col]
    slotpos = jnp.where(valid & (rank < DEG), rank * N + col, NSLOT)
    n_inval = jnp.maximum(CAP - cnt, 1)
    pad_ptr = (cnt + (jnp.arange(NSLOT) % n_inval)).astype(jnp.int32)
    ptr_flat = pad_ptr.at[slotpos].set(ar.astype(jnp.int32), mode="drop")
    ptr3 = ptr_flat.reshape(NT, NCH2, CHUNK)

    w1 = jnp.pad(W_m1.T, ((0, 4), (0, 0)))
    b1 = b_m1.reshape(1, HM)
    w2 = W_m2.T
    b2 = b_m2.reshape(1, HM)
    w3 = jnp.pad(W_m3.T, ((0, 0), (0, HM - SD)))
    b3 = jnp.pad(b_m3.reshape(1, SD), ((0, 0), (0, HM - SD)))
    wih = W_ih.T
    bih = b_ih.reshape(1, -1)
    whh = W_hh.T
    bhh = b_hh.reshape(1, -1)
    wr1 = W_r1.T
    br1 = b_r1.reshape(1, -1)
    wr2 = W_r2.T
    br2 = b_r2.reshape(1, -1)
    wd = (W_r3[0] - W_r3[1]).reshape(1, -1)
    bd = (b_r3[0] - b_r3[1]).reshape(1, 1)

    sc_gather, sc_pgather = _make_sc_calls()
    mlp, reduce, gru, readout = _make_tc_calls()

    def step(h2, _):
        hr2, hc2 = sc_gather(h2, h2, ridx3, cidx3)
        msgs = mlp(hr2, hc2, ef, w1, b1, w2, b2, w3, b3)
        slots = sc_pgather(msgs, ptr3)
        nm = reduce(slots.reshape(DEG, N, HM))
        h2 = gru(nm, h2, wih, bih, whh, bhh)
        return h2, None

    h2 = jnp.zeros((N, HM), _f32)
    h2, _ = lax.scan(step, h2, None, length=N_STEPS)
    return readout(h2[:, :SD], wr1, br1, wr2, br2, wd, bd)
